# baseline (speedup 1.0000x reference)
import functools

import jax
import jax.numpy as jnp
from jax import lax
from jax.experimental import pallas as pl
from jax.experimental.pallas import tpu as pltpu
from jax.experimental.pallas import tpu_sc as plsc

_N = 100
_HALF = 64
_D = 2 * _HALF
_LANES = 16
_ROWS_PER_W = 4
_NUM_W = _N // _ROWS_PER_W


@functools.partial(
    pl.kernel,
    out_type=jax.ShapeDtypeStruct((_N * _D,), jnp.float32),
    mesh=plsc.VectorSubcoreMesh(core_axis_name="c", subcore_axis_name="s"),
    scratch_types=[
        pltpu.VMEM((_HALF,), jnp.float32),
        pltpu.VMEM((_HALF,), jnp.float32),
        pltpu.VMEM((_ROWS_PER_W * _D,), jnp.float32),
    ],
)
def _sc_broadcast(price_hbm, size_hbm, out_hbm, price_v, size_v, rows_v):
    wid = lax.axis_index("s") * 2 + lax.axis_index("c")

    @pl.when(wid < _NUM_W)
    def _():
        pltpu.sync_copy(price_hbm, price_v)
        pltpu.sync_copy(size_hbm, size_v)
        for c in range(_HALF // _LANES):
            pv = price_v[pl.ds(c * _LANES, _LANES)]
            sv = size_v[pl.ds(c * _LANES, _LANES)]
            for r in range(_ROWS_PER_W):
                rows_v[pl.ds(r * _D + c * _LANES, _LANES)] = pv
                rows_v[pl.ds(r * _D + _HALF + c * _LANES, _LANES)] = sv
        block = _ROWS_PER_W * _D
        pltpu.sync_copy(rows_v, out_hbm.at[pl.ds(wid * block, block)])


def kernel(price_weight, size_weight, num_target_features):
    del num_target_features
    out = _sc_broadcast(price_weight.reshape(_HALF), size_weight.reshape(_HALF))
    return out.reshape(_N, _D)


# speedup vs baseline: 1.0830x; 1.0830x over previous
import functools

import jax
import jax.numpy as jnp
from jax import lax
from jax.experimental import pallas as pl
from jax.experimental.pallas import tpu as pltpu
from jax.experimental.pallas import tpu_sc as plsc

_N = 100
_HALF = 64
_D = 2 * _HALF
_LANES = 16
_ROWS_PER_W = 4
_NUM_W = _N // _ROWS_PER_W


@functools.partial(
    pl.kernel,
    out_type=jax.ShapeDtypeStruct((_N * _D,), jnp.float32),
    mesh=plsc.VectorSubcoreMesh(
        core_axis_name="c", subcore_axis_name="s", num_cores=1
    ),
    scratch_types=[
        pltpu.VMEM((_HALF,), jnp.float32),
        pltpu.VMEM((_HALF,), jnp.float32),
        pltpu.VMEM((_ROWS_PER_W * _D,), jnp.float32),
    ],
)
def _sc_broadcast(price_hbm, size_hbm, out_hbm, price_v, size_v, rows_v):
    wid = lax.axis_index("s")

    pltpu.sync_copy(price_hbm, price_v)
    pltpu.sync_copy(size_hbm, size_v)
    for c in range(_HALF // _LANES):
        pv = price_v[pl.ds(c * _LANES, _LANES)]
        sv = size_v[pl.ds(c * _LANES, _LANES)]
        for r in range(_ROWS_PER_W):
            rows_v[pl.ds(r * _D + c * _LANES, _LANES)] = pv
            rows_v[pl.ds(r * _D + _HALF + c * _LANES, _LANES)] = sv
    block = _ROWS_PER_W * _D
    for k in range(2):
        b = wid * 2 + k

        @pl.when(b < _NUM_W)
        def _():
            pltpu.sync_copy(rows_v, out_hbm.at[pl.ds(b * block, block)])


def kernel(price_weight, size_weight, num_target_features):
    del num_target_features
    out = _sc_broadcast(price_weight.reshape(_HALF), size_weight.reshape(_HALF))
    return out.reshape(_N, _D)


# speedup vs baseline: 1.1282x; 1.0417x over previous
import functools

import jax
import jax.numpy as jnp
from jax import lax
from jax.experimental import pallas as pl
from jax.experimental.pallas import tpu as pltpu
from jax.experimental.pallas import tpu_sc as plsc

_N = 100
_HALF = 64
_D = 2 * _HALF
_LANES = 16
_NW = 16
_CHUNK = _N * _D // _NW
_PAT_ROWS = _CHUNK // _D + 1


@functools.partial(
    pl.kernel,
    out_type=jax.ShapeDtypeStruct((_N * _D,), jnp.float32),
    mesh=plsc.VectorSubcoreMesh(
        core_axis_name="c", subcore_axis_name="s", num_cores=1
    ),
    scratch_types=[
        pltpu.VMEM((_HALF,), jnp.float32),
        pltpu.VMEM((_HALF,), jnp.float32),
        pltpu.VMEM((_PAT_ROWS * _D,), jnp.float32),
        pltpu.SemaphoreType.DMA,
        pltpu.SemaphoreType.DMA,
    ],
)
def _sc_broadcast(price_hbm, size_hbm, out_hbm, price_v, size_v, pat_v,
                  sem_p, sem_s):
    wid = lax.axis_index("s")
    cp = pltpu.async_copy(price_hbm, price_v, sem_p)
    cs = pltpu.async_copy(size_hbm, size_v, sem_s)
    cp.wait()
    cs.wait()
    for c in range(_HALF // _LANES):
        pv = price_v[pl.ds(c * _LANES, _LANES)]
        sv = size_v[pl.ds(c * _LANES, _LANES)]
        for r in range(_PAT_ROWS):
            pat_v[pl.ds(r * _D + c * _LANES, _LANES)] = pv
            pat_v[pl.ds(r * _D + _HALF + c * _LANES, _LANES)] = sv
    phase = pl.multiple_of(lax.rem(wid * _CHUNK, _D), 32)
    pltpu.sync_copy(
        pat_v.at[pl.ds(phase, _CHUNK)],
        out_hbm.at[pl.ds(wid * _CHUNK, _CHUNK)],
    )


def kernel(price_weight, size_weight, num_target_features):
    del num_target_features
    out = _sc_broadcast(price_weight.reshape(_HALF), size_weight.reshape(_HALF))
    return out.reshape(_N, _D)
